# baseline (speedup 1.0000x reference)
import functools

import jax
import jax.numpy as jnp
from jax import lax
from jax.experimental import pallas as pl
from jax.experimental.pallas import tpu as pltpu
from jax.experimental.pallas import tpu_sc as plsc

_N = 20000
_K = 4096
_NPOST = 500
_NMS_THRESH = 0.25
_SCORE_THRESH = 0.1
_TR = 512

_R, _C = 256, 128
_NPAD = _R * _C
_LOGN = 15

_NW = 32
_BPW = _K // _NW
_D = 8


def _partner(x, j, rows):
    s = 1 << j
    if j < 7:
        left = jnp.concatenate([x[:, s:], x[:, :s]], axis=1)
        right = jnp.concatenate([x[:, _C - s:], x[:, :_C - s]], axis=1)
        bit = (jax.lax.broadcasted_iota(jnp.int32, (rows, _C), 1) >> j) & 1
    else:
        sr = s // _C
        left = jnp.concatenate([x[sr:, :], x[:sr, :]], axis=0)
        right = jnp.concatenate([x[rows - sr:, :], x[:rows - sr, :]], axis=0)
        bit = (jax.lax.broadcasted_iota(jnp.int32, (rows, _C), 0) >> (j - 7)) & 1
    return jnp.where(bit == 0, left, right), bit


def _desc_mask(k, rows):
    if k < 7:
        bit = (jax.lax.broadcasted_iota(jnp.int32, (rows, _C), 1) >> k) & 1
    else:
        bit = (jax.lax.broadcasted_iota(jnp.int32, (rows, _C), 0) >> (k - 7)) & 1
    return bit == 0


def _cx(ka, ia, j, desc, rows):
    kb, bit = _partner(ka, j, rows)
    ib, _ = _partner(ia, j, rows)
    first = (ka > kb) | ((ka == kb) & (ia < ib))
    getwin = (bit == 0) if desc is None else jnp.logical_not(
        jnp.logical_xor(bit == 0, desc))
    keep_a = jnp.logical_not(jnp.logical_xor(first, getwin))
    return jnp.where(keep_a, ka, kb), jnp.where(keep_a, ia, ib)


def _sort_body(p_ref, okey_ref, oidx_ref):
    p = p_ref[...]
    ka = jnp.where(p >= _SCORE_THRESH, p, -jnp.inf)
    ia = (jax.lax.broadcasted_iota(jnp.int32, (_R, _C), 0) * _C
          + jax.lax.broadcasted_iota(jnp.int32, (_R, _C), 1))
    for k in range(1, 13):
        desc = _desc_mask(k, _R)
        for j in range(k - 1, -1, -1):
            ka, ia = _cx(ka, ia, j, desc, _R)
    rows = _R
    for _ in range(3):
        ka, ia = _cx(ka, ia, 12, None, rows)
        blocks = [slice(64 * c, 64 * c + 32) for c in range(rows // 64)]
        ka = jnp.concatenate([ka[sl, :] for sl in blocks], axis=0)
        ia = jnp.concatenate([ia[sl, :] for sl in blocks], axis=0)
        rows //= 2
        desc = None if rows == 32 else _desc_mask(12, rows)
        for j in range(11, -1, -1):
            ka, ia = _cx(ka, ia, j, desc, rows)
    okey_ref[...] = ka
    oidx_ref[...] = ia


def _bitonic_sort(p2d):
    return pl.pallas_call(
        _sort_body,
        out_shape=(jax.ShapeDtypeStruct((32, _C), jnp.float32),
                   jax.ShapeDtypeStruct((32, _C), jnp.int32)),
    )(p2d)


def _sc_gather_body(table_hbm, idx_hbm, out_hbm, idx_v, rows_v, sem):
    wid = lax.axis_index("s") * 2 + lax.axis_index("c")
    base = wid * _BPW
    pltpu.sync_copy(idx_hbm.at[pl.ds(base, _BPW)], idx_v)
    pltpu.async_copy(table_hbm.at[idx_v], rows_v, sem).wait()
    pltpu.sync_copy(rows_v, out_hbm.at[pl.ds(base, _BPW)])


def _sc_gather(table, idx):
    mesh = plsc.VectorSubcoreMesh(core_axis_name="c", subcore_axis_name="s")
    f = functools.partial(
        pl.kernel, mesh=mesh,
        compiler_params=pltpu.CompilerParams(use_tc_tiling_on_sc=False),
        out_type=jax.ShapeDtypeStruct((_K, _D), jnp.float32),
        scratch_types=[
            pltpu.VMEM((_BPW,), jnp.int32),
            pltpu.VMEM((_BPW, _D), jnp.float32),
            pltpu.SemaphoreType.DMA,
        ],
    )(_sc_gather_body)
    return f(table, idx)


def _sc_scatter_body(rows_hbm, tgt_hbm, out_hbm, tgt_v, rows_v, sem):
    wid = lax.axis_index("s") * 2 + lax.axis_index("c")
    base = wid * _BPW
    pltpu.sync_copy(tgt_hbm.at[pl.ds(base, _BPW)], tgt_v)
    pltpu.sync_copy(rows_hbm.at[pl.ds(base, _BPW)], rows_v)
    pltpu.async_copy(rows_v, out_hbm.at[tgt_v], sem).wait()


def _sc_scatter(rows, tgt):
    mesh = plsc.VectorSubcoreMesh(core_axis_name="c", subcore_axis_name="s")
    f = functools.partial(
        pl.kernel, mesh=mesh,
        compiler_params=pltpu.CompilerParams(use_tc_tiling_on_sc=False),
        out_type=jax.ShapeDtypeStruct((_K, _D), jnp.float32),
        scratch_types=[
            pltpu.VMEM((_BPW,), jnp.int32),
            pltpu.VMEM((_BPW, _D), jnp.float32),
            pltpu.SemaphoreType.DMA,
        ],
    )(_sc_scatter_body)
    return f(rows, tgt)


def _nms_body(b_ref, bt_ref, s_ref, v_ref, vcol_ref, rows_ref, tgt_ref, adj_ref):
    col = jax.lax.broadcasted_iota(jnp.int32, (_TR, _K), 1)

    def build_tile(t, carry):
        ts = t * _TR
        x1r = b_ref[pl.ds(ts, _TR), 0:1]
        y1r = b_ref[pl.ds(ts, _TR), 1:2]
        x2r = b_ref[pl.ds(ts, _TR), 2:3]
        y2r = b_ref[pl.ds(ts, _TR), 3:4]
        x1c = bt_ref[0:1, :]
        y1c = bt_ref[1:2, :]
        x2c = bt_ref[2:3, :]
        y2c = bt_ref[3:4, :]
        xx1 = jnp.maximum(x1r, x1c)
        yy1 = jnp.maximum(y1r, y1c)
        xx2 = jnp.minimum(x2r, x2c)
        yy2 = jnp.minimum(y2r, y2c)
        w = jnp.clip(xx2 - xx1, 0.0, None)
        h = jnp.clip(yy2 - yy1, 0.0, None)
        inter = w * h
        area_r = (x2r - x1r) * (y2r - y1r)
        area_c = (x2c - x1c) * (y2c - y1c)
        iou = inter / (area_r + area_c - inter + 1e-8)
        row = jax.lax.broadcasted_iota(jnp.int32, (_TR, _K), 0) + ts
        vrow = vcol_ref[pl.ds(ts, _TR), 0:1] > 0.0
        adj = (iou > _NMS_THRESH) & (col > row) & vrow
        adj_ref[pl.ds(ts, _TR), :] = jnp.where(adj, 1.0, 0.0).astype(jnp.bfloat16)
        return carry

    jax.lax.fori_loop(0, _K // _TR, build_tile, 0)

    v = v_ref[0:1, :] > 0.0

    def cond(carry):
        _, changed = carry
        return changed

    def body(carry):
        k, _ = carry
        m = jnp.dot(k.astype(jnp.bfloat16), adj_ref[...],
                    preferred_element_type=jnp.float32)
        nk = jnp.where(v & (m < 0.5), 1.0, 0.0)
        changed = jnp.sum(jnp.abs(nk - k)) > 0.0
        return nk, changed

    k0 = jnp.where(v, 1.0, 0.0)
    keep, _ = jax.lax.while_loop(cond, body, (k0, jnp.bool_(True)))

    for c in range(4):
        rows_ref[c:c + 1, :] = bt_ref[c:c + 1, :] * keep
    rows_ref[4:5, :] = s_ref[0:1, :] * keep
    rows_ref[5:8, :] = jnp.zeros((3, _K), jnp.float32)

    c = keep
    for d in [1, 2, 4, 8, 16, 32, 64, 128, 256, 512, 1024, 2048]:
        c = c + jnp.concatenate(
            [jnp.zeros((1, d), jnp.float32), c[:, :_K - d]], axis=1)
    pos = c - keep
    nk = c[0:1, _K - 1:_K]
    iota = jax.lax.broadcasted_iota(jnp.int32, (1, _K), 1).astype(jnp.float32)
    tgt = jnp.where(keep > 0.5, pos, nk + (iota - pos))
    tgt_ref[0:1, :] = tgt.astype(jnp.int32)


def _nms_rows_tgt(b, bt, s_row, v_row, v_col):
    return pl.pallas_call(
        _nms_body,
        out_shape=(jax.ShapeDtypeStruct((8, _K), jnp.float32),
                   jax.ShapeDtypeStruct((1, _K), jnp.int32)),
        scratch_shapes=[pltpu.VMEM((_K, _K), jnp.bfloat16)],
    )(b, bt, s_row, v_row, v_col)


def kernel(boxes, scores):
    probs = jax.nn.sigmoid(scores)
    ppad = jnp.concatenate(
        [probs, jnp.full((_NPAD - _N,), -jnp.inf, jnp.float32)])
    skey, sidx = _bitonic_sort(ppad.reshape(_R, _C))
    skey = skey.reshape(_K)
    idx = sidx.reshape(_K)
    table = jnp.concatenate(
        [boxes, jnp.zeros((_N, _D - 4), jnp.float32)], axis=1)
    b = _sc_gather(table, idx)[:, :4]
    s = jnp.maximum(skey, 0.0)
    vf = (skey >= _SCORE_THRESH).astype(jnp.float32)
    rows_t, tgt = _nms_rows_tgt(b, b.T, s.reshape(1, _K),
                                vf.reshape(1, _K), vf.reshape(_K, 1))
    scat = _sc_scatter(rows_t.T, tgt.reshape(_K))
    return scat[:_NPOST, :5]
